# baseline (speedup 1.0000x reference)
import jax
from jax import lax
import jax.numpy as jnp
from jax.experimental import pallas as pl
from jax.experimental.pallas import tpu as pltpu
from jax.experimental.pallas import tpu_sc as plsc

_SHAPE = (64, 32, 32768)
_ROWS = _SHAPE[0] * _SHAPE[1]
_COLS = _SHAPE[2]
_LANES = 16
_NW = 32
_RPW = _ROWS // _NW
_CH = 8192
_CPR = _COLS // _CH
_NCH = _RPW * _CPR
_NBUF = 4


def _sc_apply(x2, w_vec, bv_vec):
    mesh = plsc.VectorSubcoreMesh(core_axis_name="c", subcore_axis_name="s")

    @pl.kernel(
        out_type=jax.ShapeDtypeStruct((_ROWS, _COLS), jnp.float32),
        mesh=mesh,
        scratch_types=[
            pltpu.VMEM((_NBUF, _CH), jnp.float32),
            pltpu.VMEM((_NBUF, _CH), jnp.float32),
            pltpu.VMEM((_LANES,), jnp.float32),
            pltpu.VMEM((_LANES,), jnp.float32),
            pltpu.SemaphoreType.DMA((_NBUF,)),
            pltpu.SemaphoreType.DMA((_NBUF,)),
        ],
    )
    def sck(w_hbm, bv_hbm, x_hbm, o_hbm, ibuf, obuf, w_vmem, bv_vmem,
            isem, osem):
        pltpu.sync_copy(w_hbm, w_vmem)
        pltpu.sync_copy(bv_hbm, bv_vmem)
        wreg = w_vmem[...]
        breg = bv_vmem[...]

        wid = lax.axis_index("c") * 16 + lax.axis_index("s")
        row0 = wid * _RPW

        def in_copy(b, c):
            return pltpu.make_async_copy(
                x_hbm.at[row0 + c // _CPR, pl.ds((c % _CPR) * _CH, _CH)],
                ibuf.at[b], isem.at[b])

        def out_copy(b, c):
            return pltpu.make_async_copy(
                obuf.at[b],
                o_hbm.at[row0 + c // _CPR, pl.ds((c % _CPR) * _CH, _CH)],
                osem.at[b])

        for b in range(_NBUF):
            in_copy(b, b).start()

        @pl.loop(0, _NCH // _NBUF)
        def _(gi):
            for b in range(_NBUF):
                c = gi * _NBUF + b
                in_copy(b, c).wait()

                @plsc.parallel_loop(0, _CH, step=_LANES, unroll=16)
                def _(i):
                    xv = ibuf.at[b][pl.ds(i, _LANES)]
                    y = jnp.where(xv < breg, 0.0, xv * wreg)
                    obuf.at[b][pl.ds(i, _LANES)] = jnp.minimum(y, 1.0)

                @pl.when(c >= _NBUF)
                def _():
                    out_copy(b, c - _NBUF).wait()

                out_copy(b, c).start()

                @pl.when(c + _NBUF < _NCH)
                def _():
                    in_copy(b, c + _NBUF).start()

        for b in range(_NBUF):
            out_copy(b, _NCH - _NBUF + b).wait()

    return sck(w_vec, bv_vec, x2)


def kernel(x, w_inc, w_thr, a, b):
    x2 = x.reshape(_ROWS, _COLS)
    w = w_inc[0, 0] * a[0]
    bv = w_thr[0, 0] * b[0]
    w_vec = jnp.full((_LANES,), w, jnp.float32)
    bv_vec = jnp.full((_LANES,), bv, jnp.float32)
    out = _sc_apply(x2, w_vec, bv_vec)
    return out.reshape(x.shape)


# speedup vs baseline: 1.4954x; 1.4954x over previous
import jax
import jax.numpy as jnp
from jax.experimental import pallas as pl
from jax.experimental.pallas import tpu as pltpu
from jax.experimental.pallas import tpu_sc as plsc

_SHAPE = (64, 32, 32768)
_ROWS = _SHAPE[0] * _SHAPE[1]
_COLS = _SHAPE[2]
_LANES = 16
_BLK = 16384
_R_SC = 768
_R_TC = _ROWS - _R_SC
_TC_BLOCK = 64


def _sc_part(x2, w_vec, bv_vec):
    mesh = plsc.VectorSubcoreMesh(core_axis_name="c", subcore_axis_name="s")

    @pl.kernel(
        out_type=jax.ShapeDtypeStruct((_ROWS, _COLS), jnp.float32),
        mesh=mesh,
        scratch_types=[
            pltpu.VMEM((_LANES,), jnp.float32),
            pltpu.VMEM((_LANES,), jnp.float32),
        ],
    )
    def sck(w_hbm, bv_hbm, x_hbm, o_hbm, w_vmem, bv_vmem):
        pltpu.sync_copy(w_hbm, w_vmem)
        pltpu.sync_copy(bv_hbm, bv_vmem)
        wreg = w_vmem[...]
        breg = bv_vmem[...]

        def body(in_vmem, out_vmem):
            @plsc.parallel_loop(0, _BLK, step=_LANES, unroll=16)
            def _(i):
                xv = in_vmem[pl.ds(i, _LANES)]
                y = jnp.where(xv < breg, 0.0, xv * wreg)
                out_vmem[pl.ds(i, _LANES)] = jnp.minimum(y, 1.0)

        pltpu.emit_pipeline(
            body,
            grid=(_R_SC, _COLS // _BLK),
            in_specs=[pl.BlockSpec((None, _BLK), lambda i, j: (i, j))],
            out_specs=[pl.BlockSpec((None, _BLK), lambda i, j: (i, j))],
            core_axis_name=("c", "s"),
            dimension_semantics=(pltpu.PARALLEL, pltpu.PARALLEL),
        )(x_hbm, o_hbm)

    return sck(w_vec, bv_vec, x2)


def _tc_ew_kernel(alias_ref, winc_ref, wthr_ref, a_ref, b_ref, x_ref, o_ref):
    del alias_ref
    w = winc_ref[0, 0] * a_ref[0]
    bv = wthr_ref[0, 0] * b_ref[0]
    xv = x_ref[...]
    y = jnp.where(xv < bv, 0.0, xv * w)
    o_ref[...] = jnp.minimum(y, 1.0)


def _tc_part(partial_out, x2, w_inc, w_thr, a, b):
    return pl.pallas_call(
        _tc_ew_kernel,
        grid=(_R_TC // _TC_BLOCK,),
        in_specs=[
            pl.BlockSpec(memory_space=pl.ANY),
            pl.BlockSpec(memory_space=pltpu.SMEM),
            pl.BlockSpec(memory_space=pltpu.SMEM),
            pl.BlockSpec(memory_space=pltpu.SMEM),
            pl.BlockSpec(memory_space=pltpu.SMEM),
            pl.BlockSpec((_TC_BLOCK, _COLS),
                         lambda i: (i + _R_SC // _TC_BLOCK, 0)),
        ],
        out_specs=pl.BlockSpec((_TC_BLOCK, _COLS),
                               lambda i: (i + _R_SC // _TC_BLOCK, 0)),
        out_shape=jax.ShapeDtypeStruct((_ROWS, _COLS), jnp.float32),
        input_output_aliases={0: 0},
    )(partial_out, w_inc, w_thr, a, b, x2)


def kernel(x, w_inc, w_thr, a, b):
    x2 = x.reshape(_ROWS, _COLS)
    w = w_inc[0, 0] * a[0]
    bv = w_thr[0, 0] * b[0]
    w_vec = jnp.full((_LANES,), w, jnp.float32)
    bv_vec = jnp.full((_LANES,), bv, jnp.float32)
    partial = _sc_part(x2, w_vec, bv_vec)
    if _R_TC > 0:
        partial = _tc_part(partial, x2, w_inc, w_thr, a, b)
    return partial.reshape(x.shape)


# speedup vs baseline: 1.5123x; 1.0113x over previous
import jax
import jax.numpy as jnp
from jax.experimental import pallas as pl
from jax.experimental.pallas import tpu as pltpu
from jax.experimental.pallas import tpu_sc as plsc

_SHAPE = (64, 32, 32768)
_ROWS = _SHAPE[0] * _SHAPE[1]
_COLS = _SHAPE[2]
_LANES = 16
_BLK = 16384
_R_SC = 512
_R_TC = _ROWS - _R_SC
_TC_BLOCK = 64


def _sc_part(x2, w_vec, bv_vec):
    mesh = plsc.VectorSubcoreMesh(core_axis_name="c", subcore_axis_name="s")

    @pl.kernel(
        out_type=jax.ShapeDtypeStruct((_ROWS, _COLS), jnp.float32),
        mesh=mesh,
        scratch_types=[
            pltpu.VMEM((_LANES,), jnp.float32),
            pltpu.VMEM((_LANES,), jnp.float32),
        ],
    )
    def sck(w_hbm, bv_hbm, x_hbm, o_hbm, w_vmem, bv_vmem):
        pltpu.sync_copy(w_hbm, w_vmem)
        pltpu.sync_copy(bv_hbm, bv_vmem)
        wreg = w_vmem[...]
        breg = bv_vmem[...]

        def body(in_vmem, out_vmem):
            @plsc.parallel_loop(0, _BLK, step=_LANES, unroll=16)
            def _(i):
                xv = in_vmem[pl.ds(i, _LANES)]
                y = jnp.where(xv < breg, 0.0, xv * wreg)
                out_vmem[pl.ds(i, _LANES)] = jnp.minimum(y, 1.0)

        pltpu.emit_pipeline(
            body,
            grid=(_R_SC, _COLS // _BLK),
            in_specs=[pl.BlockSpec((None, _BLK), lambda i, j: (i, j))],
            out_specs=[pl.BlockSpec((None, _BLK), lambda i, j: (i, j))],
            core_axis_name=("c", "s"),
            dimension_semantics=(pltpu.PARALLEL, pltpu.PARALLEL),
        )(x_hbm, o_hbm)

    return sck(w_vec, bv_vec, x2)


def _tc_ew_kernel(alias_ref, winc_ref, wthr_ref, a_ref, b_ref, x_ref, o_ref):
    del alias_ref
    w = winc_ref[0, 0] * a_ref[0]
    bv = wthr_ref[0, 0] * b_ref[0]
    xv = x_ref[...]
    y = jnp.where(xv < bv, 0.0, xv * w)
    o_ref[...] = jnp.minimum(y, 1.0)


def _tc_part(partial_out, x2, w_inc, w_thr, a, b):
    return pl.pallas_call(
        _tc_ew_kernel,
        grid=(_R_TC // _TC_BLOCK,),
        in_specs=[
            pl.BlockSpec(memory_space=pl.ANY),
            pl.BlockSpec(memory_space=pltpu.SMEM),
            pl.BlockSpec(memory_space=pltpu.SMEM),
            pl.BlockSpec(memory_space=pltpu.SMEM),
            pl.BlockSpec(memory_space=pltpu.SMEM),
            pl.BlockSpec((_TC_BLOCK, _COLS),
                         lambda i: (i + _R_SC // _TC_BLOCK, 0)),
        ],
        out_specs=pl.BlockSpec((_TC_BLOCK, _COLS),
                               lambda i: (i + _R_SC // _TC_BLOCK, 0)),
        out_shape=jax.ShapeDtypeStruct((_ROWS, _COLS), jnp.float32),
        input_output_aliases={0: 0},
    )(partial_out, w_inc, w_thr, a, b, x2)


def kernel(x, w_inc, w_thr, a, b):
    x2 = x.reshape(_ROWS, _COLS)
    w = w_inc[0, 0] * a[0]
    bv = w_thr[0, 0] * b[0]
    w_vec = jnp.full((_LANES,), w, jnp.float32)
    bv_vec = jnp.full((_LANES,), bv, jnp.float32)
    partial = _sc_part(x2, w_vec, bv_vec)
    if _R_TC > 0:
        partial = _tc_part(partial, x2, w_inc, w_thr, a, b)
    return partial.reshape(x.shape)
